# Initial kernel scaffold; baseline (speedup 1.0000x reference)
import jax
import jax.numpy as jnp
from jax.experimental import pallas as pl


def kernel(x, edge_index, W1l, b1l, W1r, b1r, att1, bias1, g1, be1, W2l, b2l, W2r, b2r, att2, bias2, g2, be2, Wih0, Whh0, bih0, bhh0, Wih1, Whh1, bih1, bhh1, Wfc, bfc):
    raise NotImplementedError("write your pallas kernel here")


# speedup vs baseline: 1.0010x; 1.0010x over previous
import jax
import jax.numpy as jnp
from jax.experimental import pallas as pl
from jax.experimental.pallas import tpu as pltpu

B, S, NN, IN = 4, 12, 1000, 16
E = 16000
GH, HEADS, LH = 32, 4, 64


def _layernorm(x, g, b):
    mu = jnp.mean(x, axis=-1, keepdims=True)
    var = jnp.var(x, axis=-1, keepdims=True)
    return (x - mu) / jnp.sqrt(var + 1e-5) * g + b


def _segment_softmax(logits, seg, num_segments):
    m = jax.ops.segment_max(logits, seg, num_segments=num_segments)
    e = jnp.exp(logits - m[seg])
    s = jax.ops.segment_sum(e, seg, num_segments=num_segments)
    return e / (s[seg] + 1e-16)


def _gatv2(x, src, dst, Wl, bl, Wr, br, att, bias, heads, out_ch, concat, N):
    loop = jnp.arange(N, dtype=src.dtype)
    src = jnp.concatenate([src, loop])
    dst = jnp.concatenate([dst, loop])
    xl = (x @ Wl.T + bl).reshape(N, heads, out_ch)
    xr = (x @ Wr.T + br).reshape(N, heads, out_ch)
    e = jax.nn.leaky_relu(xl[src] + xr[dst], negative_slope=0.2)
    logits = jnp.sum(e * att[None], axis=-1)
    alpha = _segment_softmax(logits, dst, N)
    msg = xl[src] * alpha[..., None]
    out = jax.ops.segment_sum(msg, dst, num_segments=N)
    out = out.reshape(N, heads * out_ch) if concat else jnp.mean(out, axis=1)
    return out + bias


ROWS = B * NN
ROW_TILE = 400


def _lstm_body(li_ref, Wih0_ref, Whh0_ref, b0_ref, Wih1_ref, Whh1_ref,
               b1_ref, Wfc_ref, bfc_ref, out_ref):
    R = ROW_TILE
    wih0 = Wih0_ref[...]
    whh0 = Whh0_ref[...]
    b0 = b0_ref[...]
    wih1 = Wih1_ref[...]
    whh1 = Whh1_ref[...]
    b1 = b1_ref[...]

    def cell(xt, h, c, wih, whh, b):
        gates = (jnp.dot(xt, wih.T, preferred_element_type=jnp.float32)
                 + jnp.dot(h, whh.T, preferred_element_type=jnp.float32) + b)
        i = jax.nn.sigmoid(gates[:, 0 * LH:1 * LH])
        f = jax.nn.sigmoid(gates[:, 1 * LH:2 * LH])
        g = jnp.tanh(gates[:, 2 * LH:3 * LH])
        o = jax.nn.sigmoid(gates[:, 3 * LH:4 * LH])
        c = f * c + i * g
        h = o * jnp.tanh(c)
        return h, c

    h0 = jnp.zeros((R, LH), jnp.float32)
    c0 = jnp.zeros((R, LH), jnp.float32)
    h1 = jnp.zeros((R, LH), jnp.float32)
    c1 = jnp.zeros((R, LH), jnp.float32)
    for t in range(S):
        xt = li_ref[:, t * GH:(t + 1) * GH]
        h0, c0 = cell(xt, h0, c0, wih0, whh0, b0)
        h1, c1 = cell(h0, h1, c1, wih1, whh1, b1)
    pred = jnp.sum(h1 * Wfc_ref[...], axis=1, keepdims=True)
    out_ref[...] = pred + bfc_ref[0, 0]


def _lstm_fc(li, Wih0, Whh0, bih0, bhh0, Wih1, Whh1, bih1, bhh1, Wfc, bfc):
    b0 = (bih0 + bhh0).reshape(1, 4 * LH)
    b1 = (bih1 + bhh1).reshape(1, 4 * LH)
    grid = (ROWS // ROW_TILE,)
    wspec = lambda shape: pl.BlockSpec(shape, lambda i: (0,) * len(shape))
    return pl.pallas_call(
        _lstm_body,
        grid=grid,
        in_specs=[
            pl.BlockSpec((ROW_TILE, S * GH), lambda i: (i, 0)),
            wspec((4 * LH, GH)), wspec((4 * LH, LH)), wspec((1, 4 * LH)),
            wspec((4 * LH, LH)), wspec((4 * LH, LH)), wspec((1, 4 * LH)),
            wspec((1, LH)), wspec((1, 1)),
        ],
        out_specs=pl.BlockSpec((ROW_TILE, 1), lambda i: (i, 0)),
        out_shape=jax.ShapeDtypeStruct((ROWS, 1), jnp.float32),
    )(li, Wih0, Whh0, b0, Wih1, Whh1, b1, Wfc, bfc.reshape(1, 1))


def kernel(x, edge_index, W1l, b1l, W1r, b1r, att1, bias1, g1, be1,
           W2l, b2l, W2r, b2r, att2, bias2, g2, be2,
           Wih0, Whh0, bih0, bhh0, Wih1, Whh1, bih1, bhh1, Wfc, bfc):
    G = B * S
    N = G * NN
    xf = x.reshape(N, IN)
    offsets = (jnp.arange(G, dtype=edge_index.dtype) * NN)[:, None, None]
    be = edge_index[None, :, :] + offsets
    be = jnp.transpose(be, (1, 0, 2)).reshape(2, G * E)
    src, dst = be[0], be[1]
    h = _gatv2(xf, src, dst, W1l, b1l, W1r, b1r, att1, bias1, HEADS, GH, True, N)
    h = _layernorm(h, g1, be1)
    h = jax.nn.relu(h)
    h = _gatv2(h, src, dst, W2l, b2l, W2r, b2r, att2, bias2, 1, GH, False, N)
    h = _layernorm(h, g2, be2)
    h = h.reshape(B, S, NN, GH)
    li = jnp.transpose(h, (0, 2, 1, 3)).reshape(B * NN, S * GH)
    pred = _lstm_fc(li, Wih0, Whh0, bih0, bhh0, Wih1, Whh1, bih1, bhh1, Wfc, bfc)
    return pred.reshape(B, NN)


# speedup vs baseline: 11.4230x; 11.4117x over previous
import functools

import jax
import jax.numpy as jnp
from jax import lax
from jax.experimental import pallas as pl
from jax.experimental.pallas import tpu as pltpu
from jax.experimental.pallas import tpu_sc as plsc

B, S, NN, IN = 4, 12, 1000, 16
E = 16000
GH, HEADS, LH = 32, 4, 64

C = 32
EP = 17008
NP = 1001
NW = 32
G = B * S
N = G * NN


def _edge_stage_body(XLf, XRf, ATT, EPK, OUT, xlh, xrh, outtab, ebuf, epk,
                     attv, *, T):
    cid = lax.axis_index("c")
    sid = lax.axis_index("s")
    wid = sid * 2 + cid
    rounds = (T + NW - 1) // NW

    zeros16 = jnp.zeros((16,), jnp.float32)
    TW = 1000 * C

    pltpu.sync_copy(EPK, epk)

    def task_body(t):
        pltpu.sync_copy(XLf.at[t], xlh.at[pl.ds(0, TW)])
        pltpu.sync_copy(XRf.at[t], xrh.at[pl.ds(0, TW)])
        pltpu.sync_copy(ATT.at[t], attv)
        for i in range(NP * C // 16 - TW // 16):
            xlh[pl.ds(TW + i * 16, 16)] = zeros16
            xrh[pl.ds(TW + i * 16, 16)] = zeros16

        def zo(i, carry):
            outtab[pl.ds(i * 64, 16)] = zeros16
            outtab[pl.ds(i * 64 + 16, 16)] = zeros16
            outtab[pl.ds(i * 64 + 32, 16)] = zeros16
            outtab[pl.ds(i * 64 + 48, 16)] = zeros16
            return carry

        lax.fori_loop(0, NP * C // 64, zo, 0)

        def p1(g, mx):
            att_lo = attv[pl.ds(0, 16)]
            att_hi = attv[pl.ds(16, 16)]
            w = epk[pl.ds(g * 16, 16)]
            s32 = jnp.right_shift(w, 11) * C
            d32 = jnp.bitwise_and(w, 2047) * C
            acc = zeros16
            for c in range(C):
                a = plsc.load_gather(xlh, [s32 + c])
                b = plsc.load_gather(xrh, [d32 + c])
                z = a + b
                z = jnp.maximum(z, 0.0) + 0.2 * jnp.minimum(z, 0.0)
                asc = att_lo[c] if c < 16 else att_hi[c - 16]
                acc = acc + z * asc
            ebuf[pl.ds(g * 16, 16)] = acc
            return jnp.maximum(mx, acc)

        mx = lax.fori_loop(0, EP // 16, p1, jnp.full((16,), -1e30, jnp.float32))
        m = jnp.max(mx)

        for i in range(NP // 16 + 1):
            xrh[pl.ds(i * 16, 16)] = zeros16

        def p2(g, carry):
            w = epk[pl.ds(g * 16, 16)]
            d16 = jnp.bitwise_and(w, 2047)
            e16 = jnp.exp(ebuf[pl.ds(g * 16, 16)] - m)
            ebuf[pl.ds(g * 16, 16)] = e16
            plsc.addupdate_scatter(xrh, [d16], e16)
            return carry

        lax.fori_loop(0, EP // 16, p2, 0)

        def p3(g, carry):
            w = epk[pl.ds(g * 16, 16)]
            s32 = jnp.right_shift(w, 11) * C
            d16 = jnp.bitwise_and(w, 2047)
            d32 = d16 * C
            ss = plsc.load_gather(xrh, [d16])
            a16 = ebuf[pl.ds(g * 16, 16)] / (ss + 1e-16)
            for c in range(C):
                v = plsc.load_gather(xlh, [s32 + c]) * a16
                plsc.addupdate_scatter(outtab, [d32 + c], v)
            return carry

        lax.fori_loop(0, EP // 16, p3, 0)

        pltpu.sync_copy(outtab.at[pl.ds(0, TW)], OUT.at[t])

    def round_body(r, carry):
        t = r * NW + wid
        if T % NW == 0:
            task_body(t)
        else:
            @pl.when(t < T)
            def _():
                task_body(t)
        return carry

    lax.fori_loop(0, rounds, round_body, 0)


def _edge_stage(T, XL, XR, ATT, EPK):
    mesh = plsc.VectorSubcoreMesh(core_axis_name="c", subcore_axis_name="s")
    f = pl.kernel(
        functools.partial(_edge_stage_body, T=T),
        out_type=jax.ShapeDtypeStruct((T, 1000 * C), jnp.float32),
        mesh=mesh,
        compiler_params=pltpu.CompilerParams(needs_layout_passes=False),
        scratch_types=[
            pltpu.VMEM((NP * C,), jnp.float32),
            pltpu.VMEM((NP * C,), jnp.float32),
            pltpu.VMEM((NP * C,), jnp.float32),
            pltpu.VMEM((EP,), jnp.float32),
            pltpu.VMEM((EP,), jnp.int32),
            pltpu.VMEM((C,), jnp.float32),
        ],
    )
    out = f(XL.reshape(T, 1000 * C), XR.reshape(T, 1000 * C), ATT, EPK)
    return out.reshape(T, 1000, C)


def _proj1_body(x_ref, wl_ref, bl_ref, wr_ref, br_ref, xl_ref, xr_ref):
    xg = x_ref[...]
    xl = jnp.dot(xg, wl_ref[...].T, preferred_element_type=jnp.float32) + bl_ref[...]
    xr = jnp.dot(xg, wr_ref[...].T, preferred_element_type=jnp.float32) + br_ref[...]
    for h in range(HEADS):
        xl_ref[0, h] = xl[:, h * C:(h + 1) * C]
        xr_ref[0, h] = xr[:, h * C:(h + 1) * C]


def _proj1(xf, W1l, b1l, W1r, b1r):
    wspec = lambda shape: pl.BlockSpec(shape, lambda g: (0,) * len(shape))
    out_shape = jax.ShapeDtypeStruct((G, HEADS, NN, C), jnp.float32)
    return pl.pallas_call(
        _proj1_body,
        grid=(G,),
        in_specs=[
            pl.BlockSpec((NN, IN), lambda g: (g, 0)),
            wspec((HEADS * GH, IN)), wspec((1, HEADS * GH)),
            wspec((HEADS * GH, IN)), wspec((1, HEADS * GH)),
        ],
        out_specs=[
            pl.BlockSpec((1, HEADS, NN, C), lambda g: (g, 0, 0, 0)),
            pl.BlockSpec((1, HEADS, NN, C), lambda g: (g, 0, 0, 0)),
        ],
        out_shape=[out_shape, out_shape],
    )(xf, W1l, b1l.reshape(1, -1), W1r, b1r.reshape(1, -1))


def _mid_body(o1_ref, bias1_ref, g1_ref, be1_ref, w2l_ref, b2l_ref,
              w2r_ref, b2r_ref, xl2_ref, xr2_ref):
    h = jnp.concatenate([o1_ref[0, i] for i in range(HEADS)], axis=1)
    h = h + bias1_ref[...]
    mu = jnp.mean(h, axis=-1, keepdims=True)
    var = jnp.mean((h - mu) * (h - mu), axis=-1, keepdims=True)
    h = (h - mu) / jnp.sqrt(var + 1e-5) * g1_ref[...] + be1_ref[...]
    h = jnp.maximum(h, 0.0)
    xl2_ref[0] = jnp.dot(h, w2l_ref[...].T, preferred_element_type=jnp.float32) + b2l_ref[...]
    xr2_ref[0] = jnp.dot(h, w2r_ref[...].T, preferred_element_type=jnp.float32) + b2r_ref[...]


def _mid(out1, bias1, g1, be1, W2l, b2l, W2r, b2r):
    wspec = lambda shape: pl.BlockSpec(shape, lambda g: (0,) * len(shape))
    out_shape = jax.ShapeDtypeStruct((G, NN, C), jnp.float32)
    return pl.pallas_call(
        _mid_body,
        grid=(G,),
        in_specs=[
            pl.BlockSpec((1, HEADS, NN, C), lambda g: (g, 0, 0, 0)),
            wspec((1, HEADS * GH)), wspec((1, HEADS * GH)), wspec((1, HEADS * GH)),
            wspec((GH, HEADS * GH)), wspec((1, GH)),
            wspec((GH, HEADS * GH)), wspec((1, GH)),
        ],
        out_specs=[
            pl.BlockSpec((1, NN, C), lambda g: (g, 0, 0)),
            pl.BlockSpec((1, NN, C), lambda g: (g, 0, 0)),
        ],
        out_shape=[out_shape, out_shape],
    )(out1, bias1.reshape(1, -1), g1.reshape(1, -1), be1.reshape(1, -1),
      W2l, b2l.reshape(1, -1), W2r, b2r.reshape(1, -1))


ROW_TILE = 200


def _lstm_body(o2_ref, bias2_ref, g2_ref, be2_ref, Wih0_ref, Whh0_ref, b0_ref,
               Wih1_ref, Whh1_ref, b1_ref, Wfc_ref, bfc_ref, out_ref):
    R = ROW_TILE
    wih0 = Wih0_ref[...]
    whh0 = Whh0_ref[...]
    b0 = b0_ref[...]
    wih1 = Wih1_ref[...]
    whh1 = Whh1_ref[...]
    b1 = b1_ref[...]
    g2 = g2_ref[...]
    be2 = be2_ref[...]
    bias2 = bias2_ref[...]

    def cell(xt, h, c, wih, whh, b):
        gates = (jnp.dot(xt, wih.T, preferred_element_type=jnp.float32)
                 + jnp.dot(h, whh.T, preferred_element_type=jnp.float32) + b)
        i = jax.nn.sigmoid(gates[:, 0 * LH:1 * LH])
        f = jax.nn.sigmoid(gates[:, 1 * LH:2 * LH])
        g = jnp.tanh(gates[:, 2 * LH:3 * LH])
        o = jax.nn.sigmoid(gates[:, 3 * LH:4 * LH])
        c = f * c + i * g
        h = o * jnp.tanh(c)
        return h, c

    h0 = jnp.zeros((R, LH), jnp.float32)
    c0 = jnp.zeros((R, LH), jnp.float32)
    h1 = jnp.zeros((R, LH), jnp.float32)
    c1 = jnp.zeros((R, LH), jnp.float32)
    for t in range(S):
        xt = o2_ref[0, t] + bias2
        mu = jnp.mean(xt, axis=-1, keepdims=True)
        var = jnp.mean((xt - mu) * (xt - mu), axis=-1, keepdims=True)
        xt = (xt - mu) / jnp.sqrt(var + 1e-5) * g2 + be2
        h0, c0 = cell(xt, h0, c0, wih0, whh0, b0)
        h1, c1 = cell(h0, h1, c1, wih1, whh1, b1)
    pred = jnp.sum(h1 * Wfc_ref[...], axis=1, keepdims=True)
    out_ref[...] = pred + bfc_ref[0, 0]


def _lstm_fc(out2, bias2, g2, be2, Wih0, Whh0, bih0, bhh0,
             Wih1, Whh1, bih1, bhh1, Wfc, bfc):
    o2v = out2.reshape(B, S, NN, C)
    b0 = (bih0 + bhh0).reshape(1, 4 * LH)
    b1 = (bih1 + bhh1).reshape(1, 4 * LH)
    wspec = lambda shape: pl.BlockSpec(shape, lambda b, n: (0,) * len(shape))
    grid = (B, NN // ROW_TILE)
    return pl.pallas_call(
        _lstm_body,
        grid=grid,
        in_specs=[
            pl.BlockSpec((1, S, ROW_TILE, C), lambda b, n: (b, 0, n, 0)),
            wspec((1, C)), wspec((1, C)), wspec((1, C)),
            wspec((4 * LH, GH)), wspec((4 * LH, LH)), wspec((1, 4 * LH)),
            wspec((4 * LH, LH)), wspec((4 * LH, LH)), wspec((1, 4 * LH)),
            wspec((1, LH)), wspec((1, 1)),
        ],
        out_specs=pl.BlockSpec((ROW_TILE, 1),
                               lambda b, n: (b * (NN // ROW_TILE) + n, 0)),
        out_shape=jax.ShapeDtypeStruct((B * NN, 1), jnp.float32),
    )(o2v, bias2.reshape(1, -1), g2.reshape(1, -1), be2.reshape(1, -1),
      Wih0, Whh0, b0, Wih1, Whh1, b1, Wfc, bfc.reshape(1, 1))


def kernel(x, edge_index, W1l, b1l, W1r, b1r, att1, bias1, g1, be1,
           W2l, b2l, W2r, b2r, att2, bias2, g2, be2,
           Wih0, Whh0, bih0, bhh0, Wih1, Whh1, bih1, bhh1, Wfc, bfc):
    xf = x.reshape(N, IN)

    loop = jnp.arange(NN, dtype=jnp.int32)
    pad = EP - E - NN
    src_e = jnp.concatenate([edge_index[0].astype(jnp.int32), loop,
                             jnp.zeros((pad,), jnp.int32)])
    dst_e = jnp.concatenate([edge_index[1].astype(jnp.int32), loop,
                             jnp.full((pad,), NN, jnp.int32)])
    epk = src_e * 2048 + dst_e

    XL1, XR1 = _proj1(xf, W1l, b1l, W1r, b1r)
    T1 = G * HEADS
    ATT1 = jnp.tile(att1.astype(jnp.float32), (G, 1))
    out1 = _edge_stage(T1, XL1.reshape(T1, NN, C), XR1.reshape(T1, NN, C),
                       ATT1, epk)

    XL2, XR2 = _mid(out1.reshape(G, HEADS, NN, C), bias1, g1, be1,
                    W2l, b2l, W2r, b2r)

    ATT2 = jnp.tile(att2.astype(jnp.float32), (G, 1))
    out2 = _edge_stage(G, XL2, XR2, ATT2, epk)

    pred = _lstm_fc(out2, bias2, g2, be2, Wih0, Whh0, bih0, bhh0,
                    Wih1, Whh1, bih1, bhh1, Wfc, bfc)
    return pred.reshape(B, NN)


# speedup vs baseline: 13.5153x; 1.1832x over previous
import functools

import jax
import jax.numpy as jnp
from jax import lax
from jax.experimental import pallas as pl
from jax.experimental.pallas import tpu as pltpu
from jax.experimental.pallas import tpu_sc as plsc

B, S, NN, IN = 4, 12, 1000, 16
E = 16000
GH, HEADS, LH = 32, 4, 64

C = 32
EP = 17024
NP = 1001
NW = 32
G = B * S
N = G * NN


def _edge_stage_body(XLf, XRf, ATT, EPK, OUT, xlh, xrh, outtab, ebuf, epk,
                     attv, *, T):
    cid = lax.axis_index("c")
    sid = lax.axis_index("s")
    wid = sid * 2 + cid
    rounds = (T + NW - 1) // NW

    zeros16 = jnp.zeros((16,), jnp.float32)
    TW = 1000 * C

    pltpu.sync_copy(EPK, epk)

    def task_body(t):
        pltpu.sync_copy(XLf.at[t], xlh.at[pl.ds(0, TW)])
        pltpu.sync_copy(XRf.at[t], xrh.at[pl.ds(0, TW)])
        pltpu.sync_copy(ATT.at[t], attv)
        for i in range(NP * C // 16 - TW // 16):
            xlh[pl.ds(TW + i * 16, 16)] = zeros16
            xrh[pl.ds(TW + i * 16, 16)] = zeros16

        @plsc.parallel_loop(0, NP * C // 64)
        def _zo(i):
            outtab[pl.ds(i * 64, 16)] = zeros16
            outtab[pl.ds(i * 64 + 16, 16)] = zeros16
            outtab[pl.ds(i * 64 + 32, 16)] = zeros16
            outtab[pl.ds(i * 64 + 48, 16)] = zeros16

        def p1(g, mx):
            att_lo = attv[pl.ds(0, 16)]
            att_hi = attv[pl.ds(16, 16)]
            w0 = epk[pl.ds(g * 32, 16)]
            w1 = epk[pl.ds(g * 32 + 16, 16)]
            s0 = jnp.right_shift(w0, 11) * C
            d0 = jnp.bitwise_and(w0, 2047) * C
            s1 = jnp.right_shift(w1, 11) * C
            d1 = jnp.bitwise_and(w1, 2047) * C
            acc = [zeros16, zeros16, zeros16, zeros16]
            for c in range(C):
                a0 = plsc.load_gather(xlh, [s0 + c])
                b0 = plsc.load_gather(xrh, [d0 + c])
                a1 = plsc.load_gather(xlh, [s1 + c])
                b1 = plsc.load_gather(xrh, [d1 + c])
                z0 = a0 + b0
                z1 = a1 + b1
                z0 = jnp.maximum(z0, 0.0) + 0.2 * jnp.minimum(z0, 0.0)
                z1 = jnp.maximum(z1, 0.0) + 0.2 * jnp.minimum(z1, 0.0)
                asc = att_lo[c] if c < 16 else att_hi[c - 16]
                acc[2 * (c % 2)] = acc[2 * (c % 2)] + z0 * asc
                acc[2 * (c % 2) + 1] = acc[2 * (c % 2) + 1] + z1 * asc
            ebuf[pl.ds(g * 32, 16)] = acc[0] + acc[2]
            ebuf[pl.ds(g * 32 + 16, 16)] = acc[1] + acc[3]
            return jnp.maximum(mx, jnp.maximum(acc[0] + acc[2],
                                               acc[1] + acc[3]))

        mx = lax.fori_loop(0, EP // 32, p1,
                           jnp.full((16,), -1e30, jnp.float32))
        m = jnp.max(mx)

        for i in range(NP // 16 + 1):
            xrh[pl.ds(i * 16, 16)] = zeros16

        @plsc.parallel_loop(0, EP // 16)
        def _p2(g):
            w = epk[pl.ds(g * 16, 16)]
            d16 = jnp.bitwise_and(w, 2047)
            e16 = jnp.exp(ebuf[pl.ds(g * 16, 16)] - m)
            ebuf[pl.ds(g * 16, 16)] = e16
            plsc.addupdate_scatter(xrh, [d16], e16)

        @plsc.parallel_loop(0, EP // 16)
        def _p3(g):
            w = epk[pl.ds(g * 16, 16)]
            s32 = jnp.right_shift(w, 11) * C
            d16 = jnp.bitwise_and(w, 2047)
            d32 = d16 * C
            ss = plsc.load_gather(xrh, [d16])
            a16 = ebuf[pl.ds(g * 16, 16)] / (ss + 1e-16)
            for c in range(C):
                v = plsc.load_gather(xlh, [s32 + c]) * a16
                plsc.addupdate_scatter(outtab, [d32 + c], v)

        pltpu.sync_copy(outtab.at[pl.ds(0, TW)], OUT.at[t])

    def round_body(r, carry):
        t = r * NW + wid
        if T % NW == 0:
            task_body(t)
        else:
            @pl.when(t < T)
            def _():
                task_body(t)
        return carry

    lax.fori_loop(0, rounds, round_body, 0)


def _edge_stage(T, XL, XR, ATT, EPK):
    mesh = plsc.VectorSubcoreMesh(core_axis_name="c", subcore_axis_name="s")
    f = pl.kernel(
        functools.partial(_edge_stage_body, T=T),
        out_type=jax.ShapeDtypeStruct((T, 1000 * C), jnp.float32),
        mesh=mesh,
        compiler_params=pltpu.CompilerParams(needs_layout_passes=False),
        scratch_types=[
            pltpu.VMEM((NP * C,), jnp.float32),
            pltpu.VMEM((NP * C,), jnp.float32),
            pltpu.VMEM((NP * C,), jnp.float32),
            pltpu.VMEM((EP,), jnp.float32),
            pltpu.VMEM((EP,), jnp.int32),
            pltpu.VMEM((C,), jnp.float32),
        ],
    )
    out = f(XL.reshape(T, 1000 * C), XR.reshape(T, 1000 * C), ATT, EPK)
    return out.reshape(T, 1000, C)


def _proj1_body(x_ref, wl_ref, bl_ref, wr_ref, br_ref, xl_ref, xr_ref):
    xg = x_ref[...]
    xl = jnp.dot(xg, wl_ref[...].T, preferred_element_type=jnp.float32) + bl_ref[...]
    xr = jnp.dot(xg, wr_ref[...].T, preferred_element_type=jnp.float32) + br_ref[...]
    for h in range(HEADS):
        xl_ref[0, h] = xl[:, h * C:(h + 1) * C]
        xr_ref[0, h] = xr[:, h * C:(h + 1) * C]


def _proj1(xf, W1l, b1l, W1r, b1r):
    wspec = lambda shape: pl.BlockSpec(shape, lambda g: (0,) * len(shape))
    out_shape = jax.ShapeDtypeStruct((G, HEADS, NN, C), jnp.float32)
    return pl.pallas_call(
        _proj1_body,
        grid=(G,),
        in_specs=[
            pl.BlockSpec((NN, IN), lambda g: (g, 0)),
            wspec((HEADS * GH, IN)), wspec((1, HEADS * GH)),
            wspec((HEADS * GH, IN)), wspec((1, HEADS * GH)),
        ],
        out_specs=[
            pl.BlockSpec((1, HEADS, NN, C), lambda g: (g, 0, 0, 0)),
            pl.BlockSpec((1, HEADS, NN, C), lambda g: (g, 0, 0, 0)),
        ],
        out_shape=[out_shape, out_shape],
    )(xf, W1l, b1l.reshape(1, -1), W1r, b1r.reshape(1, -1))


def _mid_body(o1_ref, bias1_ref, g1_ref, be1_ref, w2l_ref, b2l_ref,
              w2r_ref, b2r_ref, xl2_ref, xr2_ref):
    h = jnp.concatenate([o1_ref[0, i] for i in range(HEADS)], axis=1)
    h = h + bias1_ref[...]
    mu = jnp.mean(h, axis=-1, keepdims=True)
    var = jnp.mean((h - mu) * (h - mu), axis=-1, keepdims=True)
    h = (h - mu) / jnp.sqrt(var + 1e-5) * g1_ref[...] + be1_ref[...]
    h = jnp.maximum(h, 0.0)
    xl2_ref[0] = jnp.dot(h, w2l_ref[...].T, preferred_element_type=jnp.float32) + b2l_ref[...]
    xr2_ref[0] = jnp.dot(h, w2r_ref[...].T, preferred_element_type=jnp.float32) + b2r_ref[...]


def _mid(out1, bias1, g1, be1, W2l, b2l, W2r, b2r):
    wspec = lambda shape: pl.BlockSpec(shape, lambda g: (0,) * len(shape))
    out_shape = jax.ShapeDtypeStruct((G, NN, C), jnp.float32)
    return pl.pallas_call(
        _mid_body,
        grid=(G,),
        in_specs=[
            pl.BlockSpec((1, HEADS, NN, C), lambda g: (g, 0, 0, 0)),
            wspec((1, HEADS * GH)), wspec((1, HEADS * GH)), wspec((1, HEADS * GH)),
            wspec((GH, HEADS * GH)), wspec((1, GH)),
            wspec((GH, HEADS * GH)), wspec((1, GH)),
        ],
        out_specs=[
            pl.BlockSpec((1, NN, C), lambda g: (g, 0, 0)),
            pl.BlockSpec((1, NN, C), lambda g: (g, 0, 0)),
        ],
        out_shape=[out_shape, out_shape],
    )(out1, bias1.reshape(1, -1), g1.reshape(1, -1), be1.reshape(1, -1),
      W2l, b2l.reshape(1, -1), W2r, b2r.reshape(1, -1))


ROW_TILE = 200


def _lstm_body(o2_ref, bias2_ref, g2_ref, be2_ref, Wih0_ref, Whh0_ref, b0_ref,
               Wih1_ref, Whh1_ref, b1_ref, Wfc_ref, bfc_ref, out_ref):
    R = ROW_TILE
    wih0 = Wih0_ref[...]
    whh0 = Whh0_ref[...]
    b0 = b0_ref[...]
    wih1 = Wih1_ref[...]
    whh1 = Whh1_ref[...]
    b1 = b1_ref[...]
    g2 = g2_ref[...]
    be2 = be2_ref[...]
    bias2 = bias2_ref[...]

    def cell(xt, h, c, wih, whh, b):
        gates = (jnp.dot(xt, wih.T, preferred_element_type=jnp.float32)
                 + jnp.dot(h, whh.T, preferred_element_type=jnp.float32) + b)
        i = jax.nn.sigmoid(gates[:, 0 * LH:1 * LH])
        f = jax.nn.sigmoid(gates[:, 1 * LH:2 * LH])
        g = jnp.tanh(gates[:, 2 * LH:3 * LH])
        o = jax.nn.sigmoid(gates[:, 3 * LH:4 * LH])
        c = f * c + i * g
        h = o * jnp.tanh(c)
        return h, c

    h0 = jnp.zeros((R, LH), jnp.float32)
    c0 = jnp.zeros((R, LH), jnp.float32)
    h1 = jnp.zeros((R, LH), jnp.float32)
    c1 = jnp.zeros((R, LH), jnp.float32)
    for t in range(S):
        xt = o2_ref[0, t] + bias2
        mu = jnp.mean(xt, axis=-1, keepdims=True)
        var = jnp.mean((xt - mu) * (xt - mu), axis=-1, keepdims=True)
        xt = (xt - mu) / jnp.sqrt(var + 1e-5) * g2 + be2
        h0, c0 = cell(xt, h0, c0, wih0, whh0, b0)
        h1, c1 = cell(h0, h1, c1, wih1, whh1, b1)
    pred = jnp.sum(h1 * Wfc_ref[...], axis=1, keepdims=True)
    out_ref[...] = pred + bfc_ref[0, 0]


def _lstm_fc(out2, bias2, g2, be2, Wih0, Whh0, bih0, bhh0,
             Wih1, Whh1, bih1, bhh1, Wfc, bfc):
    o2v = out2.reshape(B, S, NN, C)
    b0 = (bih0 + bhh0).reshape(1, 4 * LH)
    b1 = (bih1 + bhh1).reshape(1, 4 * LH)
    wspec = lambda shape: pl.BlockSpec(shape, lambda b, n: (0,) * len(shape))
    grid = (B, NN // ROW_TILE)
    return pl.pallas_call(
        _lstm_body,
        grid=grid,
        in_specs=[
            pl.BlockSpec((1, S, ROW_TILE, C), lambda b, n: (b, 0, n, 0)),
            wspec((1, C)), wspec((1, C)), wspec((1, C)),
            wspec((4 * LH, GH)), wspec((4 * LH, LH)), wspec((1, 4 * LH)),
            wspec((4 * LH, LH)), wspec((4 * LH, LH)), wspec((1, 4 * LH)),
            wspec((1, LH)), wspec((1, 1)),
        ],
        out_specs=pl.BlockSpec((ROW_TILE, 1),
                               lambda b, n: (b * (NN // ROW_TILE) + n, 0)),
        out_shape=jax.ShapeDtypeStruct((B * NN, 1), jnp.float32),
    )(o2v, bias2.reshape(1, -1), g2.reshape(1, -1), be2.reshape(1, -1),
      Wih0, Whh0, b0, Wih1, Whh1, b1, Wfc, bfc.reshape(1, 1))


def kernel(x, edge_index, W1l, b1l, W1r, b1r, att1, bias1, g1, be1,
           W2l, b2l, W2r, b2r, att2, bias2, g2, be2,
           Wih0, Whh0, bih0, bhh0, Wih1, Whh1, bih1, bhh1, Wfc, bfc):
    xf = x.reshape(N, IN)

    loop = jnp.arange(NN, dtype=jnp.int32)
    pad = EP - E - NN
    src_e = jnp.concatenate([edge_index[0].astype(jnp.int32), loop,
                             jnp.zeros((pad,), jnp.int32)])
    dst_e = jnp.concatenate([edge_index[1].astype(jnp.int32), loop,
                             jnp.full((pad,), NN, jnp.int32)])
    epk = src_e * 2048 + dst_e

    XL1, XR1 = _proj1(xf, W1l, b1l, W1r, b1r)
    T1 = G * HEADS
    ATT1 = jnp.tile(att1.astype(jnp.float32), (G, 1))
    out1 = _edge_stage(T1, XL1.reshape(T1, NN, C), XR1.reshape(T1, NN, C),
                       ATT1, epk)

    XL2, XR2 = _mid(out1.reshape(G, HEADS, NN, C), bias1, g1, be1,
                    W2l, b2l, W2r, b2r)

    ATT2 = jnp.tile(att2.astype(jnp.float32), (G, 1))
    out2 = _edge_stage(G, XL2, XR2, ATT2, epk)

    pred = _lstm_fc(out2, bias2, g2, be2, Wih0, Whh0, bih0, bhh0,
                    Wih1, Whh1, bih1, bhh1, Wfc, bfc)
    return pred.reshape(B, NN)


# speedup vs baseline: 79.0393x; 5.8482x over previous
import functools

import jax
import jax.numpy as jnp
from jax import lax
from jax.experimental import pallas as pl
from jax.experimental.pallas import tpu as pltpu
from jax.experimental.pallas import tpu_sc as plsc

B, S, NN, IN = 4, 12, 1000, 16
E = 16000
GH, HEADS, LH = 32, 4, 64

C = 32
RE = E + NN
EP = 17024
NW = 32
G = B * S
N = G * NN
TW = C * NN


def _edge_stage_body(XLf, XRf, ATT, EPK, OUT, xlh, xrh, outtab, ebuf, epk,
                     attv, *, T):
    cid = lax.axis_index("c")
    sid = lax.axis_index("s")
    wid = sid * 2 + cid
    rounds = (T + NW - 1) // NW

    zeros16 = jnp.zeros((16,), jnp.float32)
    iota16 = lax.iota(jnp.int32, 16)

    pltpu.sync_copy(EPK, epk)

    def task_body(t):
        pltpu.sync_copy(XLf.at[t], xlh)
        pltpu.sync_copy(XRf.at[t], xrh)
        pltpu.sync_copy(ATT.at[t], attv)

        @plsc.parallel_loop(0, TW // 64)
        def _zo(i):
            outtab[pl.ds(i * 64, 16)] = zeros16
            outtab[pl.ds(i * 64 + 16, 16)] = zeros16
            outtab[pl.ds(i * 64 + 32, 16)] = zeros16
            outtab[pl.ds(i * 64 + 48, 16)] = zeros16

        def p1(g, mx):
            att_lo = attv[pl.ds(0, 16)]
            att_hi = attv[pl.ds(16, 16)]
            w0 = epk[pl.ds(g * 32, 16)]
            w1 = epk[pl.ds(g * 32 + 16, 16)]
            s0 = jnp.right_shift(w0, 11)
            d0 = jnp.bitwise_and(w0, 2047)
            s1 = jnp.right_shift(w1, 11)
            d1 = jnp.bitwise_and(w1, 2047)
            acc = [zeros16, zeros16, zeros16, zeros16]
            for c in range(C):
                a0 = plsc.load_gather(xlh, [s0 + c * NN])
                b0 = plsc.load_gather(xrh, [d0 + c * NN])
                a1 = plsc.load_gather(xlh, [s1 + c * NN])
                b1 = plsc.load_gather(xrh, [d1 + c * NN])
                z0 = a0 + b0
                z1 = a1 + b1
                z0 = jnp.maximum(z0, 0.0) + 0.2 * jnp.minimum(z0, 0.0)
                z1 = jnp.maximum(z1, 0.0) + 0.2 * jnp.minimum(z1, 0.0)
                asc = att_lo[c] if c < 16 else att_hi[c - 16]
                acc[2 * (c % 2)] = acc[2 * (c % 2)] + z0 * asc
                acc[2 * (c % 2) + 1] = acc[2 * (c % 2) + 1] + z1 * asc
            ebuf[pl.ds(g * 32, 16)] = acc[0] + acc[2]
            ebuf[pl.ds(g * 32 + 16, 16)] = acc[1] + acc[3]
            return jnp.maximum(mx, jnp.maximum(acc[0] + acc[2],
                                               acc[1] + acc[3]))

        mx = lax.fori_loop(0, EP // 32, p1,
                           jnp.full((16,), -1e30, jnp.float32))
        m = jnp.max(mx)

        for i in range(NN // 16 + 1):
            xrh[pl.ds(i * 16, 16)] = zeros16

        @plsc.parallel_loop(0, EP // 16)
        def _p2(g):
            w = epk[pl.ds(g * 16, 16)]
            d16 = jnp.bitwise_and(w, 2047)
            e16 = jnp.exp(ebuf[pl.ds(g * 16, 16)] - m)
            ebuf[pl.ds(g * 16, 16)] = e16
            msk = (iota16 + g * 16) < RE
            plsc.addupdate_scatter(xrh, [d16], e16, mask=msk)

        @plsc.parallel_loop(0, EP // 16)
        def _p3(g):
            w = epk[pl.ds(g * 16, 16)]
            s16 = jnp.right_shift(w, 11)
            d16 = jnp.bitwise_and(w, 2047)
            ss = plsc.load_gather(xrh, [d16])
            a16 = ebuf[pl.ds(g * 16, 16)] / (ss + 1e-16)
            msk = (iota16 + g * 16) < RE
            for c in range(C):
                v = plsc.load_gather(xlh, [s16 + c * NN]) * a16
                plsc.addupdate_scatter(outtab, [d16 + c * NN], v, mask=msk)

        pltpu.sync_copy(outtab, OUT.at[t])

    def round_body(r, carry):
        t = r * NW + wid
        if T % NW == 0:
            task_body(t)
        else:
            @pl.when(t < T)
            def _():
                task_body(t)
        return carry

    lax.fori_loop(0, rounds, round_body, 0)


def _edge_stage(T, XLf, XRf, ATT, EPK):
    mesh = plsc.VectorSubcoreMesh(core_axis_name="c", subcore_axis_name="s")
    f = pl.kernel(
        functools.partial(_edge_stage_body, T=T),
        out_type=jax.ShapeDtypeStruct((T, TW), jnp.float32),
        mesh=mesh,
        compiler_params=pltpu.CompilerParams(needs_layout_passes=False),
        scratch_types=[
            pltpu.VMEM((TW,), jnp.float32),
            pltpu.VMEM((TW,), jnp.float32),
            pltpu.VMEM((TW,), jnp.float32),
            pltpu.VMEM((EP,), jnp.float32),
            pltpu.VMEM((EP,), jnp.int32),
            pltpu.VMEM((C,), jnp.float32),
        ],
    )
    return f(XLf, XRf, ATT, EPK)


_CMAJ = (((1,), (1,)), ((), ()))


def _proj1_body(x_ref, wl_ref, bl_ref, wr_ref, br_ref, xl_ref, xr_ref):
    xg = x_ref[...]
    xl = lax.dot_general(wl_ref[...], xg, _CMAJ,
                         preferred_element_type=jnp.float32) + bl_ref[...]
    xr = lax.dot_general(wr_ref[...], xg, _CMAJ,
                         preferred_element_type=jnp.float32) + br_ref[...]
    for h in range(HEADS):
        xl_ref[0, h] = xl[h * C:(h + 1) * C, :]
        xr_ref[0, h] = xr[h * C:(h + 1) * C, :]


def _proj1(xf, W1l, b1l, W1r, b1r):
    wspec = lambda shape: pl.BlockSpec(shape, lambda g: (0,) * len(shape))
    out_shape = jax.ShapeDtypeStruct((G, HEADS, C, NN), jnp.float32)
    return pl.pallas_call(
        _proj1_body,
        grid=(G,),
        in_specs=[
            pl.BlockSpec((NN, IN), lambda g: (g, 0)),
            wspec((HEADS * GH, IN)), wspec((HEADS * GH, 1)),
            wspec((HEADS * GH, IN)), wspec((HEADS * GH, 1)),
        ],
        out_specs=[
            pl.BlockSpec((1, HEADS, C, NN), lambda g: (g, 0, 0, 0)),
            pl.BlockSpec((1, HEADS, C, NN), lambda g: (g, 0, 0, 0)),
        ],
        out_shape=[out_shape, out_shape],
    )(xf, W1l, b1l.reshape(-1, 1), W1r, b1r.reshape(-1, 1))


def _mid_body(o1_ref, bias1_ref, g1_ref, be1_ref, w2l_ref, b2l_ref,
              w2r_ref, b2r_ref, xl2_ref, xr2_ref):
    h = jnp.concatenate([o1_ref[0, i] for i in range(HEADS)], axis=0)
    h = h + bias1_ref[...]
    mu = jnp.mean(h, axis=0, keepdims=True)
    var = jnp.mean((h - mu) * (h - mu), axis=0, keepdims=True)
    h = (h - mu) / jnp.sqrt(var + 1e-5) * g1_ref[...] + be1_ref[...]
    h = jnp.maximum(h, 0.0)
    xl2_ref[0] = jnp.dot(w2l_ref[...], h,
                         preferred_element_type=jnp.float32) + b2l_ref[...]
    xr2_ref[0] = jnp.dot(w2r_ref[...], h,
                         preferred_element_type=jnp.float32) + b2r_ref[...]


def _mid(out1, bias1, g1, be1, W2l, b2l, W2r, b2r):
    wspec = lambda shape: pl.BlockSpec(shape, lambda g: (0,) * len(shape))
    out_shape = jax.ShapeDtypeStruct((G, C, NN), jnp.float32)
    return pl.pallas_call(
        _mid_body,
        grid=(G,),
        in_specs=[
            pl.BlockSpec((1, HEADS, C, NN), lambda g: (g, 0, 0, 0)),
            wspec((HEADS * GH, 1)), wspec((HEADS * GH, 1)), wspec((HEADS * GH, 1)),
            wspec((GH, HEADS * GH)), wspec((GH, 1)),
            wspec((GH, HEADS * GH)), wspec((GH, 1)),
        ],
        out_specs=[
            pl.BlockSpec((1, C, NN), lambda g: (g, 0, 0)),
            pl.BlockSpec((1, C, NN), lambda g: (g, 0, 0)),
        ],
        out_shape=[out_shape, out_shape],
    )(out1, bias1.reshape(-1, 1), g1.reshape(-1, 1), be1.reshape(-1, 1),
      W2l, b2l.reshape(-1, 1), W2r, b2r.reshape(-1, 1))


ROW_TILE = NN


def _lstm_body(o2_ref, bias2_ref, g2_ref, be2_ref, Wih0_ref, Whh0_ref, b0_ref,
               Wih1_ref, Whh1_ref, b1_ref, Wfc_ref, bfc_ref, out_ref):
    R = ROW_TILE
    wih0 = Wih0_ref[...]
    whh0 = Whh0_ref[...]
    b0 = b0_ref[...]
    wih1 = Wih1_ref[...]
    whh1 = Whh1_ref[...]
    b1 = b1_ref[...]
    g2 = g2_ref[...]
    be2 = be2_ref[...]
    bias2 = bias2_ref[...]

    def cell(xt, h, c, wih, whh, b):
        gates = (jnp.dot(wih, xt, preferred_element_type=jnp.float32)
                 + jnp.dot(whh, h, preferred_element_type=jnp.float32) + b)
        i = jax.nn.sigmoid(gates[0 * LH:1 * LH, :])
        f = jax.nn.sigmoid(gates[1 * LH:2 * LH, :])
        g = jnp.tanh(gates[2 * LH:3 * LH, :])
        o = jax.nn.sigmoid(gates[3 * LH:4 * LH, :])
        c = f * c + i * g
        h = o * jnp.tanh(c)
        return h, c

    h0 = jnp.zeros((LH, R), jnp.float32)
    c0 = jnp.zeros((LH, R), jnp.float32)
    h1 = jnp.zeros((LH, R), jnp.float32)
    c1 = jnp.zeros((LH, R), jnp.float32)
    for t in range(S):
        xt = o2_ref[0, t] + bias2
        mu = jnp.mean(xt, axis=0, keepdims=True)
        var = jnp.mean((xt - mu) * (xt - mu), axis=0, keepdims=True)
        xt = (xt - mu) / jnp.sqrt(var + 1e-5) * g2 + be2
        h0, c0 = cell(xt, h0, c0, wih0, whh0, b0)
        h1, c1 = cell(h0, h1, c1, wih1, whh1, b1)
    pred = jnp.sum(h1 * Wfc_ref[...], axis=0, keepdims=True)
    out_ref[0] = pred + bfc_ref[0, 0]


def _lstm_fc(out2, bias2, g2, be2, Wih0, Whh0, bih0, bhh0,
             Wih1, Whh1, bih1, bhh1, Wfc, bfc):
    o2v = out2.reshape(B, S, C, NN)
    b0 = (bih0 + bhh0).reshape(4 * LH, 1)
    b1 = (bih1 + bhh1).reshape(4 * LH, 1)
    wspec = lambda shape: pl.BlockSpec(shape, lambda b: (0,) * len(shape))
    grid = (B,)
    return pl.pallas_call(
        _lstm_body,
        grid=grid,
        in_specs=[
            pl.BlockSpec((1, S, C, NN), lambda b: (b, 0, 0, 0)),
            wspec((C, 1)), wspec((C, 1)), wspec((C, 1)),
            wspec((4 * LH, GH)), wspec((4 * LH, LH)), wspec((4 * LH, 1)),
            wspec((4 * LH, LH)), wspec((4 * LH, LH)), wspec((4 * LH, 1)),
            wspec((LH, 1)), wspec((1, 1)),
        ],
        out_specs=pl.BlockSpec((1, 1, NN), lambda b: (b, 0, 0)),
        out_shape=jax.ShapeDtypeStruct((B, 1, NN), jnp.float32),
    )(o2v, bias2.reshape(-1, 1), g2.reshape(-1, 1), be2.reshape(-1, 1),
      Wih0, Whh0, b0, Wih1, Whh1, b1, Wfc.reshape(-1, 1),
      bfc.reshape(1, 1))


def kernel(x, edge_index, W1l, b1l, W1r, b1r, att1, bias1, g1, be1,
           W2l, b2l, W2r, b2r, att2, bias2, g2, be2,
           Wih0, Whh0, bih0, bhh0, Wih1, Whh1, bih1, bhh1, Wfc, bfc):
    xf = x.reshape(N, IN)

    loop = jnp.arange(NN, dtype=jnp.int32)
    src_e = jnp.concatenate([edge_index[0].astype(jnp.int32), loop,
                             jnp.zeros((EP - RE,), jnp.int32)])
    dst_e = jnp.concatenate([edge_index[1].astype(jnp.int32), loop,
                             jnp.zeros((EP - RE,), jnp.int32)])
    epk = src_e * 2048 + dst_e

    XL1, XR1 = _proj1(xf, W1l, b1l, W1r, b1r)
    T1 = G * HEADS
    ATT1 = jnp.tile(att1.astype(jnp.float32), (G, 1))
    out1 = _edge_stage(T1, XL1.reshape(T1, TW), XR1.reshape(T1, TW),
                       ATT1, epk)

    XL2, XR2 = _mid(out1.reshape(G, HEADS, C, NN), bias1, g1, be1,
                    W2l, b2l, W2r, b2r)

    ATT2 = jnp.tile(att2.astype(jnp.float32), (G, 1))
    out2 = _edge_stage(G, XL2.reshape(G, TW), XR2.reshape(G, TW), ATT2, epk)

    pred = _lstm_fc(out2.reshape(G, C, NN), bias2, g2, be2,
                     Wih0, Whh0, bih0, bhh0, Wih1, Whh1, bih1, bhh1, Wfc, bfc)
    return pred.reshape(B, NN)
